# baseline (speedup 1.0000x reference)
import functools
import jax
import jax.numpy as jnp
from jax import lax
from jax.experimental import pallas as pl
from jax.experimental.pallas import tpu as pltpu
from jax.experimental.pallas import tpu_sc as plsc

N = 10000
E = 320000
D = 128
H = 16
OUT = 7

NC = 2
NS = 16
NW = NC * NS
C = 1250
K = E // (NW * C)
NPAD = 10240
RS = NPAD // NS
NB = 2
G = K // NB
DEPTH1 = 6
L = 16
assert NW * K * C == E and K % NB == 0

_sc_mesh = plsc.VectorSubcoreMesh(
    core_axis_name="c", subcore_axis_name="s", num_cores=NC, num_subcores=NS)
_sc_params = pltpu.CompilerParams(use_tc_tiling_on_sc=False,
                                  needs_layout_passes=False)

_GDN = lax.GatherDimensionNumbers(
    offset_dims=(), collapsed_slice_dims=(0,), start_index_map=(0,))


def _lane_bcast(v, r):
    idx = jnp.full((L, 1), r, jnp.int32)
    return lax.gather(v, idx, _GDN, (1,),
                      mode=lax.GatherScatterMode.PROMISE_IN_BOUNDS)


def _rsqrt16(d):
    bits = plsc.bitcast(d, jnp.int32)
    y = plsc.bitcast(jnp.int32(0x5F3759DF) - (bits >> 1), jnp.float32)
    for _ in range(3):
        y = y * (1.5 - 0.5 * d * y * y)
    return y


def _ring_loop(tbl_sh, acc_sh, src_v, dst_v, rows_v, gsem, ssem):
    for b in range(NB):
        pltpu.async_copy(tbl_sh.at[src_v.at[b]], rows_v.at[b], gsem.at[b])

    def outer(g, carry):
        base = g * NB
        for b in range(NB):
            j = base + b
            pltpu.make_async_copy(tbl_sh.at[src_v.at[j]], rows_v.at[b],
                                  gsem.at[b]).wait()
            pltpu.async_copy(rows_v.at[b], acc_sh.at[dst_v.at[j]],
                             ssem.at[b], add=True)

        @pl.when(g < G - 1)
        def _():
            for b in range(NB):
                j = base + b
                pltpu.make_async_copy(rows_v.at[b], acc_sh.at[dst_v.at[j]],
                                      ssem.at[b]).wait()
                pltpu.async_copy(tbl_sh.at[src_v.at[j + NB]],
                                 rows_v.at[b], gsem.at[b])

        return carry

    lax.fori_loop(0, G, outer, 0)
    for b in range(NB):
        pltpu.make_async_copy(rows_v.at[b], acc_sh.at[dst_v.at[b]],
                              ssem.at[b]).wait()


@functools.partial(
    pl.kernel,
    out_type=jax.ShapeDtypeStruct((NC, NPAD), jnp.float32),
    mesh=_sc_mesh,
    compiler_params=_sc_params,
    scratch_types=[
        pltpu.VMEM((K, C), jnp.int32),
        pltpu.VMEM((C,), jnp.float32),
        pltpu.VMEM_SHARED((NPAD,), jnp.float32),
        pltpu.SemaphoreType.DMA,
    ],
)
def _deg_kernel(dsts_hbm, ones_c_hbm, ones_s_hbm, zeros_s_hbm, out_hbm,
                dst_v, ones_v, acc_sh, ssem):
    c = lax.axis_index("c")
    s = lax.axis_index("s")
    wid = c * NS + s
    pltpu.sync_copy(dsts_hbm.at[wid], dst_v)
    pltpu.sync_copy(ones_c_hbm, ones_v)

    @pl.when(c == 0)
    def _():
        pltpu.sync_copy(ones_s_hbm, acc_sh.at[pl.ds(s * RS, RS)])

    @pl.when(c == 1)
    def _():
        pltpu.sync_copy(zeros_s_hbm, acc_sh.at[pl.ds(s * RS, RS)])

    plsc.subcore_barrier()

    def step(j, carry):
        pltpu.async_copy(ones_v, acc_sh.at[dst_v.at[j]], ssem, add=True)

        @pl.when(j >= DEPTH1)
        def _():
            pltpu.make_async_copy(ones_v, acc_sh.at[dst_v.at[j]], ssem).wait()

        return carry

    lax.fori_loop(0, K, step, 0)
    for _ in range(min(DEPTH1, K)):
        pltpu.make_async_copy(ones_v, acc_sh.at[dst_v.at[0]], ssem).wait()
    plsc.subcore_barrier()
    pltpu.sync_copy(acc_sh.at[pl.ds(s * RS, RS)],
                    out_hbm.at[c, pl.ds(s * RS, RS)])


@functools.partial(
    pl.kernel,
    out_type=[jax.ShapeDtypeStruct((NC, NPAD, H), jnp.float32),
              jax.ShapeDtypeStruct((NPAD,), jnp.float32)],
    mesh=_sc_mesh,
    compiler_params=_sc_params,
    scratch_types=[
        pltpu.VMEM((K, C), jnp.int32),
        pltpu.VMEM((K, C), jnp.int32),
        pltpu.VMEM((NB, C, H), jnp.float32),
        pltpu.VMEM((RS, H), jnp.float32),
        pltpu.VMEM((RS,), jnp.float32),
        pltpu.VMEM((RS,), jnp.float32),
        pltpu.VMEM((RS,), jnp.float32),
        pltpu.VMEM_SHARED((NPAD, H), jnp.float32),
        pltpu.VMEM_SHARED((NPAD, H), jnp.float32),
        pltpu.SemaphoreType.DMA((NB,)),
        pltpu.SemaphoreType.DMA((NB,)),
    ],
)
def _agg1_kernel(h1_hbm, degp_hbm, srcs_hbm, dsts_hbm, zrows_hbm,
                 p_hbm, dinv_hbm,
                 src_v, dst_v, rows_v, slab_v, d0_v, d1_v, dinv_v,
                 acc_sh, tbl_sh, gsem, ssem):
    c = lax.axis_index("c")
    s = lax.axis_index("s")
    wid = c * NS + s
    stripe = pl.ds(s * RS, RS)
    pltpu.async_copy(srcs_hbm.at[wid], src_v, gsem.at[0])
    pltpu.async_copy(dsts_hbm.at[wid], dst_v, gsem.at[1])
    pltpu.async_copy(degp_hbm.at[0, stripe], d0_v, gsem.at[2])
    pltpu.async_copy(degp_hbm.at[1, stripe], d1_v, gsem.at[3])
    pltpu.async_copy(h1_hbm.at[stripe], slab_v, gsem.at[0])
    pltpu.make_async_copy(srcs_hbm.at[wid], src_v, gsem.at[0]).wait()
    pltpu.make_async_copy(dsts_hbm.at[wid], dst_v, gsem.at[1]).wait()
    pltpu.make_async_copy(degp_hbm.at[0, stripe], d0_v, gsem.at[2]).wait()
    pltpu.make_async_copy(degp_hbm.at[1, stripe], d1_v, gsem.at[3]).wait()
    pltpu.make_async_copy(h1_hbm.at[stripe], slab_v, gsem.at[0]).wait()

    def chunk(ci, carry):
        cs = pl.ds(ci * L, L)
        dv = _rsqrt16(d0_v[cs] + d1_v[cs])
        dinv_v[cs] = dv
        for r in range(L):
            i = ci * L + r
            slab_v[i] = slab_v[i] * _lane_bcast(dv, r)
        return carry

    lax.fori_loop(0, RS // L, chunk, 0)

    @pl.when(c == 0)
    def _():
        pltpu.sync_copy(dinv_v, dinv_hbm.at[stripe])
        pltpu.sync_copy(slab_v, acc_sh.at[stripe])

    @pl.when(c == 1)
    def _():
        pltpu.sync_copy(zrows_hbm, acc_sh.at[stripe])

    pltpu.sync_copy(slab_v, tbl_sh.at[stripe])
    plsc.subcore_barrier()
    _ring_loop(tbl_sh, acc_sh, src_v, dst_v, rows_v, gsem, ssem)
    plsc.subcore_barrier()
    pltpu.sync_copy(acc_sh.at[stripe], p_hbm.at[c, stripe])


@functools.partial(
    pl.kernel,
    out_type=jax.ShapeDtypeStruct((NC, NPAD, H), jnp.float32),
    mesh=_sc_mesh,
    compiler_params=_sc_params,
    scratch_types=[
        pltpu.VMEM((K, C), jnp.int32),
        pltpu.VMEM((K, C), jnp.int32),
        pltpu.VMEM((NB, C, H), jnp.float32),
        pltpu.VMEM((RS, H), jnp.float32),
        pltpu.VMEM((RS, H), jnp.float32),
        pltpu.VMEM((RS,), jnp.float32),
        pltpu.VMEM((L,), jnp.float32),
        pltpu.VMEM_SHARED((NPAD, H), jnp.float32),
        pltpu.VMEM_SHARED((NPAD, H), jnp.float32),
        pltpu.SemaphoreType.DMA((NB,)),
        pltpu.SemaphoreType.DMA((NB,)),
    ],
)
def _agg2_kernel(p_hbm, dinv_hbm, b1_hbm, srcs_hbm, dsts_hbm, zrows_hbm,
                 q_hbm,
                 src_v, dst_v, rows_v, slab0_v, slab1_v, dinv_v, b1_v,
                 acc_sh, tbl_sh, gsem, ssem):
    c = lax.axis_index("c")
    s = lax.axis_index("s")
    wid = c * NS + s
    stripe = pl.ds(s * RS, RS)
    pltpu.async_copy(srcs_hbm.at[wid], src_v, gsem.at[0])
    pltpu.async_copy(dsts_hbm.at[wid], dst_v, gsem.at[1])
    pltpu.async_copy(p_hbm.at[0, stripe], slab0_v, gsem.at[2])
    pltpu.async_copy(p_hbm.at[1, stripe], slab1_v, gsem.at[3])
    pltpu.async_copy(dinv_hbm.at[stripe], dinv_v, gsem.at[4])
    pltpu.async_copy(b1_hbm, b1_v, gsem.at[0])
    pltpu.make_async_copy(srcs_hbm.at[wid], src_v, gsem.at[0]).wait()
    pltpu.make_async_copy(dsts_hbm.at[wid], dst_v, gsem.at[1]).wait()
    pltpu.make_async_copy(p_hbm.at[0, stripe], slab0_v, gsem.at[2]).wait()
    pltpu.make_async_copy(p_hbm.at[1, stripe], slab1_v, gsem.at[3]).wait()
    pltpu.make_async_copy(dinv_hbm.at[stripe], dinv_v, gsem.at[4]).wait()
    pltpu.make_async_copy(b1_hbm, b1_v, gsem.at[0]).wait()
    b1vec = b1_v[...]

    def chunk(ci, carry):
        dv = dinv_v[pl.ds(ci * L, L)]
        for r in range(L):
            i = ci * L + r
            sc = _lane_bcast(dv, r)
            z = (slab0_v[i] + slab1_v[i]) * sc + b1vec
            slab0_v[i] = jnp.maximum(z, 0.0) * sc
        return carry

    lax.fori_loop(0, RS // L, chunk, 0)

    @pl.when(c == 0)
    def _():
        pltpu.sync_copy(slab0_v, acc_sh.at[stripe])

    @pl.when(c == 1)
    def _():
        pltpu.sync_copy(zrows_hbm, acc_sh.at[stripe])

    pltpu.sync_copy(slab0_v, tbl_sh.at[stripe])
    plsc.subcore_barrier()
    _ring_loop(tbl_sh, acc_sh, src_v, dst_v, rows_v, gsem, ssem)
    plsc.subcore_barrier()

    pltpu.sync_copy(acc_sh.at[stripe], slab0_v)

    def pchunk(ci, carry):
        dv = dinv_v[pl.ds(ci * L, L)]
        for r in range(L):
            i = ci * L + r
            slab0_v[i] = slab0_v[i] * _lane_bcast(dv, r)
        return carry

    lax.fori_loop(0, RS // L, pchunk, 0)
    pltpu.sync_copy(slab0_v, q_hbm.at[c, stripe])


def _mm1_body(x_ref, w1_ref, h1_ref):
    h1_ref[:N, :] = jnp.dot(x_ref[...], w1_ref[...],
                            preferred_element_type=jnp.float32)
    h1_ref[N:, :] = jnp.zeros((NPAD - N, H), jnp.float32)


def _out_body(q_ref, b2_ref, w2_ref, o_ref):
    t = q_ref[0, :N] + q_ref[1, :N]
    o = jnp.dot(t, w2_ref[...], preferred_element_type=jnp.float32) + b2_ref[...]
    m = jnp.max(o, axis=1, keepdims=True)
    lse = jnp.log(jnp.sum(jnp.exp(o - m), axis=1, keepdims=True)) + m
    o_ref[...] = o - lse


def kernel(x, edge_index, W1, b1, W2, b2):
    ei = edge_index.astype(jnp.int32)
    srcs = ei[0].reshape(NW, K, C)
    dsts = ei[1].reshape(NW, K, C)
    ones_c = jnp.ones((C,), jnp.float32)
    ones_s = jnp.ones((RS,), jnp.float32)
    zeros_s = jnp.zeros((RS,), jnp.float32)
    zrows = jnp.zeros((RS, H), jnp.float32)
    b2r = b2.reshape(1, OUT)

    degp = _deg_kernel(dsts, ones_c, ones_s, zeros_s)

    h1 = pl.pallas_call(
        _mm1_body,
        out_shape=jax.ShapeDtypeStruct((NPAD, H), jnp.float32),
    )(x, W1)

    p, dinv = _agg1_kernel(h1, degp, srcs, dsts, zrows)

    q = _agg2_kernel(p, dinv, b1, srcs, dsts, zrows)

    o = pl.pallas_call(
        _out_body,
        out_shape=jax.ShapeDtypeStruct((N, OUT), jnp.float32),
    )(q, b2r, W2)

    return o


# speedup vs baseline: 1.0271x; 1.0271x over previous
import functools
import jax
import jax.numpy as jnp
from jax import lax
from jax.experimental import pallas as pl
from jax.experimental.pallas import tpu as pltpu
from jax.experimental.pallas import tpu_sc as plsc

N = 10000
E = 320000
D = 128
H = 16
OUT = 7

NC = 2
NS = 16
NW = NC * NS
C = 1000
K = E // (NW * C)
NPAD = 10240
RS = NPAD // NS
NB = 2
G = K // NB
DEPTH1 = 6
L = 16
assert NW * K * C == E and K % NB == 0

_sc_mesh = plsc.VectorSubcoreMesh(
    core_axis_name="c", subcore_axis_name="s", num_cores=NC, num_subcores=NS)
_sc_params = pltpu.CompilerParams(use_tc_tiling_on_sc=False,
                                  needs_layout_passes=False)

_GDN = lax.GatherDimensionNumbers(
    offset_dims=(), collapsed_slice_dims=(0,), start_index_map=(0,))


def _lane_bcast(v, r):
    idx = jnp.full((L, 1), r, jnp.int32)
    return lax.gather(v, idx, _GDN, (1,),
                      mode=lax.GatherScatterMode.PROMISE_IN_BOUNDS)


def _rsqrt16(d):
    bits = plsc.bitcast(d, jnp.int32)
    y = plsc.bitcast(jnp.int32(0x5F3759DF) - (bits >> 1), jnp.float32)
    for _ in range(3):
        y = y * (1.5 - 0.5 * d * y * y)
    return y


def _ring_loop(tbl_sh, acc_sh, src_v, dst_v, rows_v, gsem, ssem):
    for b in range(NB):
        pltpu.async_copy(tbl_sh.at[src_v.at[b]], rows_v.at[b], gsem.at[b])

    def outer(g, carry):
        base = g * NB
        for b in range(NB):
            j = base + b
            pltpu.make_async_copy(tbl_sh.at[src_v.at[j]], rows_v.at[b],
                                  gsem.at[b]).wait()
            pltpu.async_copy(rows_v.at[b], acc_sh.at[dst_v.at[j]],
                             ssem.at[b], add=True)

        @pl.when(g < G - 1)
        def _():
            for b in range(NB):
                j = base + b
                pltpu.make_async_copy(rows_v.at[b], acc_sh.at[dst_v.at[j]],
                                      ssem.at[b]).wait()
                pltpu.async_copy(tbl_sh.at[src_v.at[j + NB]],
                                 rows_v.at[b], gsem.at[b])

        return carry

    lax.fori_loop(0, G, outer, 0)
    for b in range(NB):
        pltpu.make_async_copy(rows_v.at[b], acc_sh.at[dst_v.at[b]],
                              ssem.at[b]).wait()


@functools.partial(
    pl.kernel,
    out_type=jax.ShapeDtypeStruct((NC, NPAD), jnp.float32),
    mesh=_sc_mesh,
    compiler_params=_sc_params,
    scratch_types=[
        pltpu.VMEM((K, C), jnp.int32),
        pltpu.VMEM((C,), jnp.float32),
        pltpu.VMEM_SHARED((NPAD,), jnp.float32),
        pltpu.SemaphoreType.DMA,
    ],
)
def _deg_kernel(dsts_hbm, ones_c_hbm, ones_s_hbm, zeros_s_hbm, out_hbm,
                dst_v, ones_v, acc_sh, ssem):
    c = lax.axis_index("c")
    s = lax.axis_index("s")
    wid = c * NS + s
    pltpu.sync_copy(dsts_hbm.at[wid], dst_v)
    pltpu.sync_copy(ones_c_hbm, ones_v)

    @pl.when(c == 0)
    def _():
        pltpu.sync_copy(ones_s_hbm, acc_sh.at[pl.ds(s * RS, RS)])

    @pl.when(c == 1)
    def _():
        pltpu.sync_copy(zeros_s_hbm, acc_sh.at[pl.ds(s * RS, RS)])

    plsc.subcore_barrier()

    def step(j, carry):
        pltpu.async_copy(ones_v, acc_sh.at[dst_v.at[j]], ssem, add=True)

        @pl.when(j >= DEPTH1)
        def _():
            pltpu.make_async_copy(ones_v, acc_sh.at[dst_v.at[j]], ssem).wait()

        return carry

    lax.fori_loop(0, K, step, 0)
    for _ in range(min(DEPTH1, K)):
        pltpu.make_async_copy(ones_v, acc_sh.at[dst_v.at[0]], ssem).wait()
    plsc.subcore_barrier()
    pltpu.sync_copy(acc_sh.at[pl.ds(s * RS, RS)],
                    out_hbm.at[c, pl.ds(s * RS, RS)])


@functools.partial(
    pl.kernel,
    out_type=[jax.ShapeDtypeStruct((NC, NPAD, H), jnp.float32),
              jax.ShapeDtypeStruct((NPAD,), jnp.float32)],
    mesh=_sc_mesh,
    compiler_params=_sc_params,
    scratch_types=[
        pltpu.VMEM((K, C), jnp.int32),
        pltpu.VMEM((K, C), jnp.int32),
        pltpu.VMEM((NB, C, H), jnp.float32),
        pltpu.VMEM((RS, H), jnp.float32),
        pltpu.VMEM((RS,), jnp.float32),
        pltpu.VMEM((RS,), jnp.float32),
        pltpu.VMEM((RS,), jnp.float32),
        pltpu.VMEM_SHARED((NPAD, H), jnp.float32),
        pltpu.VMEM_SHARED((NPAD, H), jnp.float32),
        pltpu.SemaphoreType.DMA((NB,)),
        pltpu.SemaphoreType.DMA((NB,)),
    ],
)
def _agg1_kernel(h1_hbm, degp_hbm, srcs_hbm, dsts_hbm, zrows_hbm,
                 p_hbm, dinv_hbm,
                 src_v, dst_v, rows_v, slab_v, d0_v, d1_v, dinv_v,
                 acc_sh, tbl_sh, gsem, ssem):
    c = lax.axis_index("c")
    s = lax.axis_index("s")
    wid = c * NS + s
    stripe = pl.ds(s * RS, RS)
    pltpu.async_copy(srcs_hbm.at[wid], src_v, gsem.at[0])
    pltpu.async_copy(dsts_hbm.at[wid], dst_v, gsem.at[1])
    pltpu.async_copy(degp_hbm.at[0, stripe], d0_v, gsem.at[2])
    pltpu.async_copy(degp_hbm.at[1, stripe], d1_v, gsem.at[3])
    pltpu.async_copy(h1_hbm.at[stripe], slab_v, gsem.at[0])
    pltpu.make_async_copy(srcs_hbm.at[wid], src_v, gsem.at[0]).wait()
    pltpu.make_async_copy(dsts_hbm.at[wid], dst_v, gsem.at[1]).wait()
    pltpu.make_async_copy(degp_hbm.at[0, stripe], d0_v, gsem.at[2]).wait()
    pltpu.make_async_copy(degp_hbm.at[1, stripe], d1_v, gsem.at[3]).wait()
    pltpu.make_async_copy(h1_hbm.at[stripe], slab_v, gsem.at[0]).wait()

    def chunk(ci, carry):
        cs = pl.ds(ci * L, L)
        dv = _rsqrt16(d0_v[cs] + d1_v[cs])
        dinv_v[cs] = dv
        for r in range(L):
            i = ci * L + r
            slab_v[i] = slab_v[i] * _lane_bcast(dv, r)
        return carry

    lax.fori_loop(0, RS // L, chunk, 0)

    @pl.when(c == 0)
    def _():
        pltpu.sync_copy(dinv_v, dinv_hbm.at[stripe])
        pltpu.sync_copy(slab_v, acc_sh.at[stripe])

    @pl.when(c == 1)
    def _():
        pltpu.sync_copy(zrows_hbm, acc_sh.at[stripe])

    pltpu.sync_copy(slab_v, tbl_sh.at[stripe])
    plsc.subcore_barrier()
    _ring_loop(tbl_sh, acc_sh, src_v, dst_v, rows_v, gsem, ssem)
    plsc.subcore_barrier()
    pltpu.sync_copy(acc_sh.at[stripe], p_hbm.at[c, stripe])


@functools.partial(
    pl.kernel,
    out_type=jax.ShapeDtypeStruct((NC, NPAD, H), jnp.float32),
    mesh=_sc_mesh,
    compiler_params=_sc_params,
    scratch_types=[
        pltpu.VMEM((K, C), jnp.int32),
        pltpu.VMEM((K, C), jnp.int32),
        pltpu.VMEM((NB, C, H), jnp.float32),
        pltpu.VMEM((RS, H), jnp.float32),
        pltpu.VMEM((RS, H), jnp.float32),
        pltpu.VMEM((RS,), jnp.float32),
        pltpu.VMEM((L,), jnp.float32),
        pltpu.VMEM_SHARED((NPAD, H), jnp.float32),
        pltpu.VMEM_SHARED((NPAD, H), jnp.float32),
        pltpu.SemaphoreType.DMA((NB,)),
        pltpu.SemaphoreType.DMA((NB,)),
    ],
)
def _agg2_kernel(p_hbm, dinv_hbm, b1_hbm, srcs_hbm, dsts_hbm, zrows_hbm,
                 q_hbm,
                 src_v, dst_v, rows_v, slab0_v, slab1_v, dinv_v, b1_v,
                 acc_sh, tbl_sh, gsem, ssem):
    c = lax.axis_index("c")
    s = lax.axis_index("s")
    wid = c * NS + s
    stripe = pl.ds(s * RS, RS)
    pltpu.async_copy(srcs_hbm.at[wid], src_v, gsem.at[0])
    pltpu.async_copy(dsts_hbm.at[wid], dst_v, gsem.at[1])
    pltpu.async_copy(p_hbm.at[0, stripe], slab0_v, gsem.at[2])
    pltpu.async_copy(p_hbm.at[1, stripe], slab1_v, gsem.at[3])
    pltpu.async_copy(dinv_hbm.at[stripe], dinv_v, gsem.at[4])
    pltpu.async_copy(b1_hbm, b1_v, gsem.at[0])
    pltpu.make_async_copy(srcs_hbm.at[wid], src_v, gsem.at[0]).wait()
    pltpu.make_async_copy(dsts_hbm.at[wid], dst_v, gsem.at[1]).wait()
    pltpu.make_async_copy(p_hbm.at[0, stripe], slab0_v, gsem.at[2]).wait()
    pltpu.make_async_copy(p_hbm.at[1, stripe], slab1_v, gsem.at[3]).wait()
    pltpu.make_async_copy(dinv_hbm.at[stripe], dinv_v, gsem.at[4]).wait()
    pltpu.make_async_copy(b1_hbm, b1_v, gsem.at[0]).wait()
    b1vec = b1_v[...]

    def chunk(ci, carry):
        dv = dinv_v[pl.ds(ci * L, L)]
        for r in range(L):
            i = ci * L + r
            sc = _lane_bcast(dv, r)
            z = (slab0_v[i] + slab1_v[i]) * sc + b1vec
            slab0_v[i] = jnp.maximum(z, 0.0) * sc
        return carry

    lax.fori_loop(0, RS // L, chunk, 0)

    @pl.when(c == 0)
    def _():
        pltpu.sync_copy(slab0_v, acc_sh.at[stripe])

    @pl.when(c == 1)
    def _():
        pltpu.sync_copy(zrows_hbm, acc_sh.at[stripe])

    pltpu.sync_copy(slab0_v, tbl_sh.at[stripe])
    plsc.subcore_barrier()
    _ring_loop(tbl_sh, acc_sh, src_v, dst_v, rows_v, gsem, ssem)
    plsc.subcore_barrier()

    pltpu.sync_copy(acc_sh.at[stripe], slab0_v)

    def pchunk(ci, carry):
        dv = dinv_v[pl.ds(ci * L, L)]
        for r in range(L):
            i = ci * L + r
            slab0_v[i] = slab0_v[i] * _lane_bcast(dv, r)
        return carry

    lax.fori_loop(0, RS // L, pchunk, 0)
    pltpu.sync_copy(slab0_v, q_hbm.at[c, stripe])


def _mm1_body(x_ref, w1_ref, h1_ref):
    h1_ref[:N, :] = jnp.dot(x_ref[...], w1_ref[...],
                            preferred_element_type=jnp.float32)
    h1_ref[N:, :] = jnp.zeros((NPAD - N, H), jnp.float32)


def _out_body(q_ref, b2_ref, w2_ref, o_ref):
    t = q_ref[0, :N] + q_ref[1, :N]
    o = jnp.dot(t, w2_ref[...], preferred_element_type=jnp.float32) + b2_ref[...]
    m = jnp.max(o, axis=1, keepdims=True)
    lse = jnp.log(jnp.sum(jnp.exp(o - m), axis=1, keepdims=True)) + m
    o_ref[...] = o - lse


def kernel(x, edge_index, W1, b1, W2, b2):
    ei = edge_index.astype(jnp.int32)
    srcs = ei[0].reshape(NW, K, C)
    dsts = ei[1].reshape(NW, K, C)
    ones_c = jnp.ones((C,), jnp.float32)
    ones_s = jnp.ones((RS,), jnp.float32)
    zeros_s = jnp.zeros((RS,), jnp.float32)
    zrows = jnp.zeros((RS, H), jnp.float32)
    b2r = b2.reshape(1, OUT)

    degp = _deg_kernel(dsts, ones_c, ones_s, zeros_s)

    h1 = pl.pallas_call(
        _mm1_body,
        out_shape=jax.ShapeDtypeStruct((NPAD, H), jnp.float32),
    )(x, W1)

    p, dinv = _agg1_kernel(h1, degp, srcs, dsts, zrows)

    q = _agg2_kernel(p, dinv, b1, srcs, dsts, zrows)

    o = pl.pallas_call(
        _out_body,
        out_shape=jax.ShapeDtypeStruct((N, OUT), jnp.float32),
    )(q, b2r, W2)

    return o
